# baseline (speedup 1.0000x reference)
import jax
import jax.numpy as jnp
from jax.experimental import pallas as pl
from jax.experimental.pallas import tpu as pltpu

_NBUF = 4
_CH = 4


def _make_body(B, R, C, CH, NBUF):
    steps = B // CH
    G = steps // NBUF

    def body(p_hbm, t_hbm, o_hbm, tv, *scr):
        inbs = scr[0:NBUF]
        outbs = scr[NBUF:2 * NBUF]
        tsem = scr[2 * NBUF]
        insems = scr[2 * NBUF + 1: 3 * NBUF + 1]
        outsems = scr[3 * NBUF + 1: 4 * NBUF + 1]

        pltpu.make_async_copy(t_hbm, tv, tsem).start()
        for k in range(NBUF):
            pltpu.make_async_copy(
                p_hbm.at[pl.ds(k * CH, CH)], inbs[k], insems[k]
            ).start(priority=k % 2)
        pltpu.make_async_copy(t_hbm, tv, tsem).wait()

        def group(g, _):
            for k in range(NBUF):
                i = g * NBUF + k
                pltpu.make_async_copy(
                    p_hbm.at[pl.ds(i * CH, CH)], inbs[k], insems[k]
                ).wait()

                @pl.when(g > 0)
                def _wait_out():
                    pltpu.make_async_copy(
                        outbs[k], o_hbm.at[pl.ds(0, CH)], outsems[k]
                    ).wait()

                outbs[k][...] = inbs[k][...] + tv[None]
                pltpu.make_async_copy(
                    outbs[k], o_hbm.at[pl.ds(i * CH, CH)], outsems[k]
                ).start(priority=k % 2)

                @pl.when(g < G - 1)
                def _prefetch():
                    ni = (g + 1) * NBUF + k
                    pltpu.make_async_copy(
                        p_hbm.at[pl.ds(ni * CH, CH)], inbs[k], insems[k]
                    ).start(priority=k % 2)

            return 0

        jax.lax.fori_loop(0, G, group, 0)
        for k in range(NBUF):
            pltpu.make_async_copy(
                outbs[k], o_hbm.at[pl.ds(0, CH)], outsems[k]
            ).wait()

    return body


def kernel(patch, pos_table):
    B, P, D = patch.shape
    PD = P * D
    R = 8
    C = PD // R
    patch3 = patch.reshape(B, R, C)
    table3 = pos_table.reshape(R, C)
    scratch = (
        [pltpu.VMEM((R, C), jnp.float32)]
        + [pltpu.VMEM((_CH, R, C), jnp.float32) for _ in range(2 * _NBUF)]
        + [pltpu.SemaphoreType.DMA for _ in range(2 * _NBUF + 1)]
    )
    out = pl.pallas_call(
        _make_body(B, R, C, _CH, _NBUF),
        in_specs=[
            pl.BlockSpec(memory_space=pltpu.HBM),
            pl.BlockSpec(memory_space=pltpu.HBM),
        ],
        out_specs=pl.BlockSpec(memory_space=pltpu.HBM),
        out_shape=jax.ShapeDtypeStruct((B, R, C), patch.dtype),
        scratch_shapes=scratch,
    )(patch3, table3)
    return out.reshape(B, P, D)
